# baseline (device time: 67152 ns/iter reference)
import jax
import jax.numpy as jnp
from jax import lax
from jax.experimental import pallas as pl
from jax.experimental.pallas import tpu as pltpu

B = 8
H = 8
D = 128
BS = 16
NB = 512
P_LOC = 512
R = 8
P_REP = P_LOC // R
T_REP = P_REP * BS
PK = D + 2


def kernel(Q, K, V, bt, lens):
    lens2 = lens.reshape(B, 1)
    K2 = K.reshape(P_LOC * BS, H * D)
    V2 = V.reshape(P_LOC * BS, H * D)

    def body(q_ref, k_hbm, v_hbm, bt_ref, lens_ref, out_ref,
             kc_buf, vc_buf, part_send, part_recv, part_g, gather_buf,
             kv_sems, y_send_sem, y_recv_sem, g_send_sems, g_recv_sems,
             self_sem):
        my_x = lax.axis_index("x")
        my_y = lax.axis_index("y")
        my_z = lax.axis_index("z")
        my_r = my_x * 4 + my_z
        partner = (my_x, 1 - my_y, my_z)
        t0 = my_r * T_REP

        barrier = pltpu.get_barrier_semaphore()
        pl.semaphore_signal(barrier, inc=1, device_id=partner,
                            device_id_type=pl.DeviceIdType.MESH)
        for idx in range(R - 1):
            rr = lax.rem(my_r + 1 + idx, R)
            pl.semaphore_signal(barrier, inc=1,
                                device_id=(rr // 4, my_y, lax.rem(rr, 4)),
                                device_id_type=pl.DeviceIdType.MESH)

        kc = pltpu.make_async_copy(
            k_hbm.at[pl.ds(t0, T_REP)], kc_buf, kv_sems.at[0])
        vc = pltpu.make_async_copy(
            v_hbm.at[pl.ds(t0, T_REP)], vc_buf, kv_sems.at[1])
        kc.start()
        vc.start()

        j_iota = lax.broadcasted_iota(jnp.int32, (B, NB), 1)
        valid = j_iota < lens_ref[...]
        bt_m = jnp.where(valid, bt_ref[...], -1)
        sel = (bt_m >> 6 == my_y * 8 + my_r).astype(jnp.float32)
        lo_iota = lax.broadcasted_iota(jnp.int32, (B, NB, P_REP), 2)
        match_lo = ((bt_m & 63)[:, :, None] == lo_iota).astype(jnp.float32)
        w = (match_lo * sel[:, :, None]).sum(axis=1)
        wt = jnp.broadcast_to(w[:, :, None], (B, P_REP, BS)).reshape(B, T_REP)
        wt64 = jnp.broadcast_to(wt[:, None, :], (B, H, T_REP)).reshape(
            B * H, T_REP)

        qs = q_ref[...][:, 0]
        eye4 = (lax.broadcasted_iota(jnp.int32, (1, H, H, 1), 1)
                == lax.broadcasted_iota(jnp.int32, (1, H, H, 1), 2)
                ).astype(jnp.float32)
        qbig = (qs[:, :, None, :] * eye4).reshape(B * H, H * D)
        scale = D ** -0.5

        kc.wait()
        s = lax.dot_general(
            qbig, kc_buf[...], (((1,), (1,)), ((), ())),
            preferred_element_type=jnp.float32) * scale
        m64 = s.max(axis=1, keepdims=True)
        e = jnp.exp(s - m64) * wt64
        l64 = e.sum(axis=1, keepdims=True)
        vc.wait()
        obig = lax.dot_general(
            e, vc_buf[...], (((1,), (0,)), ((), ())),
            preferred_element_type=jnp.float32)
        o = (obig.reshape(B, H, H, D) * eye4).sum(axis=2)
        m = m64.reshape(B, H)
        l = l64.reshape(B, H)

        part_send[...] = jnp.concatenate(
            [o, m[:, :, None], l[:, :, None]], axis=2)

        pl.semaphore_wait(barrier, R)

        rdma_y = pltpu.make_async_remote_copy(
            src_ref=part_send, dst_ref=part_recv,
            send_sem=y_send_sem, recv_sem=y_recv_sem,
            device_id=partner, device_id_type=pl.DeviceIdType.MESH)
        rdma_y.start()
        rdma_y.wait_recv()

        m_p = part_recv[:, :, D]
        l_p = part_recv[:, :, D + 1]
        o_p = part_recv[:, :, :D]
        m2 = jnp.maximum(m, m_p)
        c_s = jnp.exp(m - m2)
        c_p = jnp.exp(m_p - m2)
        l2 = l * c_s + l_p * c_p
        o2 = o * c_s[:, :, None] + o_p * c_p[:, :, None]
        part_g[...] = jnp.concatenate(
            [o2, m2[:, :, None], l2[:, :, None]], axis=2)

        pltpu.make_async_copy(part_g, gather_buf.at[my_r], self_sem).start()
        for idx in range(R - 1):
            rr = lax.rem(my_r + 1 + idx, R)
            pltpu.make_async_remote_copy(
                src_ref=part_g, dst_ref=gather_buf.at[my_r],
                send_sem=g_send_sems.at[idx], recv_sem=g_recv_sems.at[my_r],
                device_id=(rr // 4, my_y, lax.rem(rr, 4)),
                device_id_type=pl.DeviceIdType.MESH).start()
        pltpu.make_async_copy(part_g, gather_buf.at[my_r], self_sem).wait()
        for idx in range(R - 1):
            rr = lax.rem(my_r + 1 + idx, R)
            pltpu.make_async_remote_copy(
                src_ref=part_g, dst_ref=gather_buf.at[rr],
                send_sem=g_send_sems.at[idx], recv_sem=g_recv_sems.at[rr],
                device_id=(rr // 4, my_y, lax.rem(rr, 4)),
                device_id_type=pl.DeviceIdType.MESH).wait_recv()

        parts = gather_buf[...]
        m_all = parts[:, :, :, D]
        l_all = parts[:, :, :, D + 1]
        o_all = parts[:, :, :, :D]
        m_g = m_all.max(axis=0)
        c = jnp.exp(m_all - m_g[None])
        l_g = (l_all * c).sum(axis=0)
        o_g = (o_all * c[:, :, :, None]).sum(axis=0)
        out_ref[...] = (o_g / l_g[:, :, None]).reshape(B, 1, H, D)

        rdma_y.wait_send()
        for idx in range(R - 1):
            rr = lax.rem(my_r + 1 + idx, R)
            pltpu.make_async_remote_copy(
                src_ref=part_g, dst_ref=gather_buf.at[my_r],
                send_sem=g_send_sems.at[idx], recv_sem=g_recv_sems.at[my_r],
                device_id=(rr // 4, my_y, lax.rem(rr, 4)),
                device_id_type=pl.DeviceIdType.MESH).wait_send()

    return pl.pallas_call(
        body,
        out_shape=jax.ShapeDtypeStruct((B, 1, H, D), jnp.float32),
        in_specs=[
            pl.BlockSpec(memory_space=pltpu.VMEM),
            pl.BlockSpec(memory_space=pltpu.MemorySpace.HBM),
            pl.BlockSpec(memory_space=pltpu.MemorySpace.HBM),
            pl.BlockSpec(memory_space=pltpu.VMEM),
            pl.BlockSpec(memory_space=pltpu.VMEM),
        ],
        out_specs=pl.BlockSpec(memory_space=pltpu.VMEM),
        scratch_shapes=[
            pltpu.VMEM((T_REP, H * D), jnp.float32),
            pltpu.VMEM((T_REP, H * D), jnp.float32),
            pltpu.VMEM((B, H, PK), jnp.float32),
            pltpu.VMEM((B, H, PK), jnp.float32),
            pltpu.VMEM((B, H, PK), jnp.float32),
            pltpu.VMEM((R, B, H, PK), jnp.float32),
            pltpu.SemaphoreType.DMA((2,)),
            pltpu.SemaphoreType.DMA,
            pltpu.SemaphoreType.DMA,
            pltpu.SemaphoreType.DMA((R - 1,)),
            pltpu.SemaphoreType.DMA((R,)),
            pltpu.SemaphoreType.DMA,
        ],
        compiler_params=pltpu.CompilerParams(collective_id=0),
    )(Q, K2, V2, bt, lens2)


# device time: 22133 ns/iter; 3.0340x vs baseline; 3.0340x over previous
import jax
import jax.numpy as jnp
from jax import lax
from jax.experimental import pallas as pl
from jax.experimental.pallas import tpu as pltpu

B = 8
H = 8
D = 128
BS = 16
NB = 512
P_LOC = 512
R = 8
P_REP = P_LOC // R
T_REP = P_REP * BS
PK = D + 2


def kernel(Q, K, V, bt, lens):
    lens2 = lens.reshape(B, 1)

    def body(q_ref, k_hbm, v_hbm, bt_ref, lens_ref, out_ref,
             kc_buf, vc_buf, part_send, part_recv, part_g, gather_buf,
             kv_sems, y_send_sem, y_recv_sem, g_send_sems, g_recv_sems,
             self_sem):
        my_x = lax.axis_index("x")
        my_y = lax.axis_index("y")
        my_z = lax.axis_index("z")
        my_r = my_x * 4 + my_z
        partner = (my_x, 1 - my_y, my_z)
        p0 = my_r * P_REP

        barrier = pltpu.get_barrier_semaphore()
        pl.semaphore_signal(barrier, inc=1, device_id=partner,
                            device_id_type=pl.DeviceIdType.MESH)
        for idx in range(R - 1):
            rr = lax.rem(my_r + 1 + idx, R)
            pl.semaphore_signal(barrier, inc=1,
                                device_id=(rr // 4, my_y, lax.rem(rr, 4)),
                                device_id_type=pl.DeviceIdType.MESH)

        kc = pltpu.make_async_copy(
            k_hbm.at[pl.ds(p0, P_REP)], kc_buf, kv_sems.at[0])
        vc = pltpu.make_async_copy(
            v_hbm.at[pl.ds(p0, P_REP)], vc_buf, kv_sems.at[1])
        kc.start()
        vc.start()

        j_iota = lax.broadcasted_iota(jnp.int32, (B, NB), 1)
        valid = j_iota < lens_ref[...]
        bt_m = jnp.where(valid, bt_ref[...], -1)
        sel = (bt_m >> 6 == my_y * 8 + my_r).astype(jnp.float32)
        lo_iota = lax.broadcasted_iota(jnp.int32, (B, NB, P_REP), 2)
        match_lo = ((bt_m & 63)[:, :, None] == lo_iota).astype(jnp.float32)
        w = (match_lo * sel[:, :, None]).sum(axis=1)
        wt = jnp.broadcast_to(w[:, :, None], (B, P_REP, BS)).reshape(B, T_REP)
        wt64 = jnp.broadcast_to(wt[:, None, :], (B, H, T_REP)).reshape(
            B * H, T_REP)

        qs = q_ref[...][:, 0]
        eye4 = (lax.broadcasted_iota(jnp.int32, (1, H, H, 1), 1)
                == lax.broadcasted_iota(jnp.int32, (1, H, H, 1), 2)
                ).astype(jnp.float32)
        qbig = (qs[:, :, None, :] * eye4).reshape(B * H, H * D)
        scale = D ** -0.5

        kc.wait()
        kc2 = kc_buf[...].reshape(T_REP, H * D)
        s = lax.dot_general(
            qbig, kc2, (((1,), (1,)), ((), ())),
            preferred_element_type=jnp.float32) * scale
        m64 = s.max(axis=1, keepdims=True)
        e = jnp.exp(s - m64) * wt64
        l64 = e.sum(axis=1, keepdims=True)
        vc.wait()
        vc3 = vc_buf[...].reshape(T_REP, H, D)
        obig = lax.dot_general(
            e, vc3, (((1,), (0,)), ((), ())),
            preferred_element_type=jnp.float32)
        o = (obig.reshape(B, H, H, D) * eye4).sum(axis=2)
        m = m64.reshape(B, H)
        l = l64.reshape(B, H)

        part_send[...] = jnp.concatenate(
            [o, m[:, :, None], l[:, :, None]], axis=2)

        pl.semaphore_wait(barrier, R)

        rdma_y = pltpu.make_async_remote_copy(
            src_ref=part_send, dst_ref=part_recv,
            send_sem=y_send_sem, recv_sem=y_recv_sem,
            device_id=partner, device_id_type=pl.DeviceIdType.MESH)
        rdma_y.start()
        rdma_y.wait_recv()

        m_p = part_recv[:, :, D]
        l_p = part_recv[:, :, D + 1]
        o_p = part_recv[:, :, :D]
        m2 = jnp.maximum(m, m_p)
        c_s = jnp.exp(m - m2)
        c_p = jnp.exp(m_p - m2)
        l2 = l * c_s + l_p * c_p
        o2 = o * c_s[:, :, None] + o_p * c_p[:, :, None]
        part_g[...] = jnp.concatenate(
            [o2, m2[:, :, None], l2[:, :, None]], axis=2)

        pltpu.make_async_copy(part_g, gather_buf.at[my_r], self_sem).start()
        for idx in range(R - 1):
            rr = lax.rem(my_r + 1 + idx, R)
            pltpu.make_async_remote_copy(
                src_ref=part_g, dst_ref=gather_buf.at[my_r],
                send_sem=g_send_sems.at[idx], recv_sem=g_recv_sems.at[my_r],
                device_id=(rr // 4, my_y, lax.rem(rr, 4)),
                device_id_type=pl.DeviceIdType.MESH).start()
        pltpu.make_async_copy(part_g, gather_buf.at[my_r], self_sem).wait()
        for idx in range(R - 1):
            rr = lax.rem(my_r + 1 + idx, R)
            pltpu.make_async_remote_copy(
                src_ref=part_g, dst_ref=gather_buf.at[rr],
                send_sem=g_send_sems.at[idx], recv_sem=g_recv_sems.at[rr],
                device_id=(rr // 4, my_y, lax.rem(rr, 4)),
                device_id_type=pl.DeviceIdType.MESH).wait_recv()

        parts = gather_buf[...]
        m_all = parts[:, :, :, D]
        l_all = parts[:, :, :, D + 1]
        o_all = parts[:, :, :, :D]
        m_g = m_all.max(axis=0)
        c = jnp.exp(m_all - m_g[None])
        l_g = (l_all * c).sum(axis=0)
        o_g = (o_all * c[:, :, :, None]).sum(axis=0)
        out_ref[...] = (o_g / l_g[:, :, None]).reshape(B, 1, H, D)

        rdma_y.wait_send()
        for idx in range(R - 1):
            rr = lax.rem(my_r + 1 + idx, R)
            pltpu.make_async_remote_copy(
                src_ref=part_g, dst_ref=gather_buf.at[my_r],
                send_sem=g_send_sems.at[idx], recv_sem=g_recv_sems.at[my_r],
                device_id=(rr // 4, my_y, lax.rem(rr, 4)),
                device_id_type=pl.DeviceIdType.MESH).wait_send()

    return pl.pallas_call(
        body,
        out_shape=jax.ShapeDtypeStruct((B, 1, H, D), jnp.float32),
        in_specs=[
            pl.BlockSpec(memory_space=pltpu.VMEM),
            pl.BlockSpec(memory_space=pltpu.MemorySpace.HBM),
            pl.BlockSpec(memory_space=pltpu.MemorySpace.HBM),
            pl.BlockSpec(memory_space=pltpu.VMEM),
            pl.BlockSpec(memory_space=pltpu.VMEM),
        ],
        out_specs=pl.BlockSpec(memory_space=pltpu.VMEM),
        scratch_shapes=[
            pltpu.VMEM((P_REP, BS, H, D), jnp.float32),
            pltpu.VMEM((P_REP, BS, H, D), jnp.float32),
            pltpu.VMEM((B, H, PK), jnp.float32),
            pltpu.VMEM((B, H, PK), jnp.float32),
            pltpu.VMEM((B, H, PK), jnp.float32),
            pltpu.VMEM((R, B, H, PK), jnp.float32),
            pltpu.SemaphoreType.DMA((2,)),
            pltpu.SemaphoreType.DMA,
            pltpu.SemaphoreType.DMA,
            pltpu.SemaphoreType.DMA((R - 1,)),
            pltpu.SemaphoreType.DMA((R,)),
            pltpu.SemaphoreType.DMA,
        ],
        compiler_params=pltpu.CompilerParams(collective_id=0),
    )(Q, K, V, bt, lens2)
